# baseline (speedup 1.0000x reference)
import functools

import jax
import jax.numpy as jnp
from jax import lax
from jax.experimental import pallas as pl
from jax.experimental.pallas import tpu as pltpu
from jax.experimental.pallas import tpu_sc as plsc

_CHUNK_ROWS = 16
_NBUF = 3


def kernel(x, emb_table):
    seq_len = x.shape[1]
    hidden = emb_table.shape[1]

    info = plsc.get_sparse_core_info()
    num_cores, num_subcores = info.num_cores, info.num_subcores
    num_workers = num_cores * num_subcores
    rows_per_w = seq_len // num_workers
    assert rows_per_w * num_workers == seq_len
    chunk = min(_CHUNK_ROWS, rows_per_w)
    if rows_per_w == 128 and chunk == 16:
        sizes = [8, 8] + [16] * 6 + [8, 8]
    else:
        sizes = [chunk] * (rows_per_w // chunk)
    assert sum(sizes) == rows_per_w
    offs = [0]
    for s in sizes:
        offs.append(offs[-1] + s)
    n_chunks = len(sizes)
    nbuf = min(_NBUF, n_chunks)

    mesh = plsc.VectorSubcoreMesh(core_axis_name="c", subcore_axis_name="s")

    @functools.partial(
        pl.kernel,
        mesh=mesh,
        out_type=jax.ShapeDtypeStruct((1, seq_len, hidden), jnp.float32),
        scratch_types=[
            pltpu.VMEM((nbuf, chunk, hidden), jnp.float32),
            pltpu.SemaphoreType.DMA((nbuf,)),
            pltpu.SemaphoreType.DMA((nbuf,)),
        ],
    )
    def copy_k(table_hbm, out_hbm, bufs, sr, sw):
        wid = lax.axis_index("s") * num_cores + lax.axis_index("c")
        base = wid * rows_per_w

        def start_read(i):
            return pltpu.async_copy(
                table_hbm.at[pl.ds(base + offs[i], sizes[i])],
                bufs.at[i % nbuf, pl.ds(0, sizes[i])],
                sr.at[i % nbuf],
            )

        def start_write(i):
            return pltpu.async_copy(
                bufs.at[i % nbuf, pl.ds(0, sizes[i])],
                out_hbm.at[0, pl.ds(base + offs[i], sizes[i])],
                sw.at[i % nbuf],
            )

        reads = [None] * nbuf
        writes = [None] * nbuf
        for i in range(min(nbuf - 1, n_chunks)):
            reads[i % nbuf] = start_read(i)
        for i in range(n_chunks):
            b = i % nbuf
            j = i + nbuf - 1
            if j < n_chunks:
                b2 = j % nbuf
                if writes[b2] is not None:
                    writes[b2].wait()
                reads[b2] = start_read(j)
            reads[b].wait()
            writes[b] = start_write(i)
        for b in range(nbuf):
            if writes[b] is not None:
                writes[b].wait()

    return copy_k(emb_table)


# speedup vs baseline: 1.0216x; 1.0216x over previous
import functools

import jax
import jax.numpy as jnp
from jax import lax
from jax.experimental import pallas as pl
from jax.experimental.pallas import tpu as pltpu
from jax.experimental.pallas import tpu_sc as plsc

_CHUNK_ROWS = 16
_NBUF = 3


def kernel(x, emb_table):
    seq_len = x.shape[1]
    hidden = emb_table.shape[1]

    info = plsc.get_sparse_core_info()
    num_cores, num_subcores = info.num_cores, info.num_subcores
    num_workers = num_cores * num_subcores
    rows_per_w = seq_len // num_workers
    assert rows_per_w * num_workers == seq_len
    chunk = min(_CHUNK_ROWS, rows_per_w)
    if rows_per_w == 128 and chunk == 16:
        sizes = [8] + [16] * 7 + [8]
    else:
        sizes = [chunk] * (rows_per_w // chunk)
    assert sum(sizes) == rows_per_w
    offs = [0]
    for s in sizes:
        offs.append(offs[-1] + s)
    n_chunks = len(sizes)
    nbuf = min(_NBUF, n_chunks)

    mesh = plsc.VectorSubcoreMesh(core_axis_name="c", subcore_axis_name="s")

    @functools.partial(
        pl.kernel,
        mesh=mesh,
        out_type=jax.ShapeDtypeStruct((1, seq_len, hidden), jnp.float32),
        scratch_types=[
            pltpu.VMEM((nbuf, chunk, hidden), jnp.float32),
            pltpu.SemaphoreType.DMA((nbuf,)),
            pltpu.SemaphoreType.DMA((nbuf,)),
        ],
    )
    def copy_k(table_hbm, out_hbm, bufs, sr, sw):
        wid = lax.axis_index("s") * num_cores + lax.axis_index("c")
        base = wid * rows_per_w

        def start_read(i):
            return pltpu.async_copy(
                table_hbm.at[pl.ds(base + offs[i], sizes[i])],
                bufs.at[i % nbuf, pl.ds(0, sizes[i])],
                sr.at[i % nbuf],
            )

        def start_write(i):
            return pltpu.async_copy(
                bufs.at[i % nbuf, pl.ds(0, sizes[i])],
                out_hbm.at[0, pl.ds(base + offs[i], sizes[i])],
                sw.at[i % nbuf],
            )

        reads = [None] * nbuf
        writes = [None] * nbuf
        for i in range(min(nbuf - 1, n_chunks)):
            reads[i % nbuf] = start_read(i)
        for i in range(n_chunks):
            b = i % nbuf
            j = i + nbuf - 1
            if j < n_chunks:
                b2 = j % nbuf
                if writes[b2] is not None:
                    writes[b2].wait()
                reads[b2] = start_read(j)
            reads[b].wait()
            writes[b] = start_write(i)
        for b in range(nbuf):
            if writes[b] is not None:
                writes[b].wait()

    return copy_k(emb_table)
